# baseline (speedup 1.0000x reference)
import jax
import jax.numpy as jnp
from jax.experimental import pallas as pl
from jax.experimental.pallas import tpu as pltpu

N_CODEBOOKS = 4
K = 1024
D = 256
BATCH = 16384
BETA = 0.25
BLK = 512
GRID = BATCH // BLK


def _rvq_body(x_ref, e0_ref, e1_ref, e2_ref, e3_ref,
              q_ref, codes_ref, loss_ref, unused_ref, counts_ref,
              en_ref, c0_ref, c1_ref, c2_ref, c3_ref):
    step = pl.program_id(0)
    table_refs = (e0_ref, e1_ref, e2_ref, e3_ref)
    cat_refs = (c0_ref, c1_ref, c2_ref, c3_ref)

    @pl.when(step == 0)
    def _init():
        counts_ref[...] = jnp.zeros_like(counts_ref)
        loss_ref[...] = jnp.zeros_like(loss_ref)
        unused_ref[...] = jnp.zeros_like(unused_ref)
        for s in range(N_CODEBOOKS):
            t = table_refs[s][...]
            en_ref[s:s + 1, :] = jnp.sum(t * t, axis=1, keepdims=True).T
            t1 = t.astype(jnp.bfloat16)
            r1 = t - t1.astype(jnp.float32)
            t2 = r1.astype(jnp.bfloat16)
            r2 = r1 - t2.astype(jnp.float32)
            t3 = r2.astype(jnp.bfloat16)
            cat_refs[s][:, 0 * D:1 * D] = t1
            cat_refs[s][:, 1 * D:2 * D] = t2
            cat_refs[s][:, 2 * D:3 * D] = t3

    x = x_ref[...]
    r = x
    q = jnp.zeros_like(x)
    loss = jnp.float32(0.0)
    codes = []
    for s, t_ref in enumerate(table_refs):
        t = t_ref[...]
        rn = jnp.sum(r * r, axis=1, keepdims=True)
        en = en_ref[s:s + 1, :]
        m2 = jax.lax.dot_general(r + r, t, (((1,), (1,)), ((), ())),
                                 preferred_element_type=jnp.float32)
        dist = (rn - m2) + en
        minv = jnp.min(dist, axis=1, keepdims=True)
        iota = jax.lax.broadcasted_iota(jnp.int32, dist.shape, 1)
        ind = jnp.min(jnp.where(dist == minv, iota, K),
                      axis=1, keepdims=True)
        hit = iota == ind
        oh16 = hit.astype(jnp.bfloat16)
        xq3 = jax.lax.dot_general(oh16, cat_refs[s][...],
                                  (((1,), (0,)), ((), ())),
                                  preferred_element_type=jnp.float32)
        xq = ((xq3[:, 0 * D:1 * D] + xq3[:, 1 * D:2 * D])
              + xq3[:, 2 * D:3 * D])
        counts_ref[s:s + 1, :] += jnp.sum(hit.astype(jnp.float32),
                                          axis=0, keepdims=True)
        loss = loss + jnp.sum(minv)
        q = q + xq
        r = r - xq
        codes.append(ind)

    q_ref[...] = q
    codes_ref[...] = jnp.concatenate(codes, axis=1)
    loss_ref[...] = loss_ref[...] + loss

    @pl.when(step == GRID - 1)
    def _finalize():
        loss_ref[...] = (loss_ref[...] * (1.0 + BETA)
                         / (N_CODEBOOKS * BATCH * D))
        unused_ref[...] = jnp.sum(
            (counts_ref[...] == 0.0).astype(jnp.int32),
            axis=(0, 1), keepdims=True)


def kernel(x, embed_0, embed_1, embed_2, embed_3):
    table_spec = pl.BlockSpec((K, D), lambda i: (0, 0))
    q, codes, loss, unused, _counts = pl.pallas_call(
        _rvq_body,
        grid=(GRID,),
        in_specs=[
            pl.BlockSpec((BLK, D), lambda i: (i, 0)),
            table_spec, table_spec, table_spec, table_spec,
        ],
        out_specs=[
            pl.BlockSpec((BLK, D), lambda i: (i, 0)),
            pl.BlockSpec((BLK, N_CODEBOOKS), lambda i: (i, 0)),
            pl.BlockSpec((1, 1), lambda i: (0, 0)),
            pl.BlockSpec((1, 1), lambda i: (0, 0)),
            pl.BlockSpec((N_CODEBOOKS, K), lambda i: (0, 0)),
        ],
        out_shape=[
            jax.ShapeDtypeStruct((BATCH, D), jnp.float32),
            jax.ShapeDtypeStruct((BATCH, N_CODEBOOKS), jnp.int32),
            jax.ShapeDtypeStruct((1, 1), jnp.float32),
            jax.ShapeDtypeStruct((1, 1), jnp.int32),
            jax.ShapeDtypeStruct((N_CODEBOOKS, K), jnp.float32),
        ],
        scratch_shapes=([pltpu.VMEM((N_CODEBOOKS, K), jnp.float32)]
                        + [pltpu.VMEM((K, 3 * D), jnp.bfloat16)] * 4),
        compiler_params=pltpu.CompilerParams(
            dimension_semantics=("arbitrary",)),
    )(x, embed_0, embed_1, embed_2, embed_3)
    return q, loss.reshape(()), unused.reshape(()), codes


# speedup vs baseline: 1.4317x; 1.4317x over previous
import jax
import jax.numpy as jnp
from jax.experimental import pallas as pl
from jax.experimental.pallas import tpu as pltpu

N_CODEBOOKS = 4
K = 1024
D = 256
BATCH = 16384
BETA = 0.25
BLK = 512
GRID = BATCH // BLK


def _rvq_body(x_ref, e0_ref, e1_ref, e2_ref, e3_ref,
              q_ref, codes_ref, loss_ref, unused_ref, counts_ref,
              en_ref, c0_ref, c1_ref, c2_ref, c3_ref):
    step = pl.program_id(0)
    table_refs = (e0_ref, e1_ref, e2_ref, e3_ref)
    cat_refs = (c0_ref, c1_ref, c2_ref, c3_ref)

    @pl.when(step == 0)
    def _init():
        counts_ref[...] = jnp.zeros_like(counts_ref)
        loss_ref[...] = jnp.zeros_like(loss_ref)
        unused_ref[...] = jnp.zeros_like(unused_ref)
        for s in range(N_CODEBOOKS):
            t = table_refs[s][...]
            en_ref[s:s + 1, :] = jnp.sum(t * t, axis=1, keepdims=True).T
            t1 = t.astype(jnp.bfloat16)
            r1 = t - t1.astype(jnp.float32)
            t2 = r1.astype(jnp.bfloat16)
            r2 = r1 - t2.astype(jnp.float32)
            t3 = r2.astype(jnp.bfloat16)
            cat_refs[s][:, 0 * D:1 * D] = t1
            cat_refs[s][:, 1 * D:2 * D] = t2
            cat_refs[s][:, 2 * D:3 * D] = t3

    x = x_ref[...]
    H = BLK // 2
    loss = jnp.float32(0.0)
    halves = []
    for h in range(2):
        rh = x[h * H:(h + 1) * H, :]
        halves.append({"r": rh, "q": jnp.zeros_like(rh), "codes": []})
    for s in range(N_CODEBOOKS):
        t = table_refs[s][...]
        en = en_ref[s:s + 1, :]
        cnt = None
        for hv in halves:
            r = hv["r"]
            rn = jnp.sum(r * r, axis=1, keepdims=True)
            m2 = jax.lax.dot_general(r + r, t, (((1,), (1,)), ((), ())),
                                     preferred_element_type=jnp.float32)
            dist = (rn - m2) + en
            minv = jnp.min(dist, axis=1, keepdims=True)
            iota = jax.lax.broadcasted_iota(jnp.int32, dist.shape, 1)
            ind = jnp.min(jnp.where(dist == minv, iota, K),
                          axis=1, keepdims=True)
            hit = iota == ind
            oh16 = hit.astype(jnp.bfloat16)
            xq3 = jax.lax.dot_general(oh16, cat_refs[s][...],
                                      (((1,), (0,)), ((), ())),
                                      preferred_element_type=jnp.float32)
            xq = ((xq3[:, 0 * D:1 * D] + xq3[:, 1 * D:2 * D])
                  + xq3[:, 2 * D:3 * D])
            c = jnp.sum(hit.astype(jnp.float32), axis=0, keepdims=True)
            cnt = c if cnt is None else cnt + c
            loss = loss + jnp.sum(minv)
            hv["q"] = hv["q"] + xq
            hv["r"] = r - xq
            hv["codes"].append(ind)
        counts_ref[s:s + 1, :] += cnt

    q_ref[...] = jnp.concatenate([hv["q"] for hv in halves], axis=0)
    codes_ref[...] = jnp.concatenate(
        [jnp.concatenate(hv["codes"], axis=1) for hv in halves], axis=0)
    loss_ref[...] = loss_ref[...] + loss

    @pl.when(step == GRID - 1)
    def _finalize():
        loss_ref[...] = (loss_ref[...] * (1.0 + BETA)
                         / (N_CODEBOOKS * BATCH * D))
        unused_ref[...] = jnp.sum(
            (counts_ref[...] == 0.0).astype(jnp.int32),
            axis=(0, 1), keepdims=True)


def kernel(x, embed_0, embed_1, embed_2, embed_3):
    table_spec = pl.BlockSpec((K, D), lambda i: (0, 0))
    q, codes, loss, unused, _counts = pl.pallas_call(
        _rvq_body,
        grid=(GRID,),
        in_specs=[
            pl.BlockSpec((BLK, D), lambda i: (i, 0)),
            table_spec, table_spec, table_spec, table_spec,
        ],
        out_specs=[
            pl.BlockSpec((BLK, D), lambda i: (i, 0)),
            pl.BlockSpec((BLK, N_CODEBOOKS), lambda i: (i, 0)),
            pl.BlockSpec((1, 1), lambda i: (0, 0)),
            pl.BlockSpec((1, 1), lambda i: (0, 0)),
            pl.BlockSpec((N_CODEBOOKS, K), lambda i: (0, 0)),
        ],
        out_shape=[
            jax.ShapeDtypeStruct((BATCH, D), jnp.float32),
            jax.ShapeDtypeStruct((BATCH, N_CODEBOOKS), jnp.int32),
            jax.ShapeDtypeStruct((1, 1), jnp.float32),
            jax.ShapeDtypeStruct((1, 1), jnp.int32),
            jax.ShapeDtypeStruct((N_CODEBOOKS, K), jnp.float32),
        ],
        scratch_shapes=([pltpu.VMEM((N_CODEBOOKS, K), jnp.float32)]
                        + [pltpu.VMEM((K, 3 * D), jnp.bfloat16)] * 4),
        compiler_params=pltpu.CompilerParams(
            dimension_semantics=("arbitrary",)),
    )(x, embed_0, embed_1, embed_2, embed_3)
    return q, loss.reshape(()), unused.reshape(()), codes


# speedup vs baseline: 1.5558x; 1.0867x over previous
import jax
import jax.numpy as jnp
from jax.experimental import pallas as pl
from jax.experimental.pallas import tpu as pltpu

N_CODEBOOKS = 4
K = 1024
D = 256
BATCH = 16384
BETA = 0.25
BLK = 2048
GRID = BATCH // BLK
NSUB = 4


def _rvq_body(x_ref, e0_ref, e1_ref, e2_ref, e3_ref,
              q_ref, codes_ref, loss_ref, unused_ref, counts_ref,
              en_ref, c0_ref, c1_ref, c2_ref, c3_ref):
    step = pl.program_id(0)
    table_refs = (e0_ref, e1_ref, e2_ref, e3_ref)
    cat_refs = (c0_ref, c1_ref, c2_ref, c3_ref)

    @pl.when(step == 0)
    def _init():
        counts_ref[...] = jnp.zeros_like(counts_ref)
        loss_ref[...] = jnp.zeros_like(loss_ref)
        unused_ref[...] = jnp.zeros_like(unused_ref)
        for s in range(N_CODEBOOKS):
            t = table_refs[s][...]
            en_ref[s:s + 1, :] = jnp.sum(t * t, axis=1, keepdims=True).T
            t1 = t.astype(jnp.bfloat16)
            r1 = t - t1.astype(jnp.float32)
            t2 = r1.astype(jnp.bfloat16)
            r2 = r1 - t2.astype(jnp.float32)
            t3 = r2.astype(jnp.bfloat16)
            cat_refs[s][:, 0 * D:1 * D] = t1
            cat_refs[s][:, 1 * D:2 * D] = t2
            cat_refs[s][:, 2 * D:3 * D] = t3

    x = x_ref[...]
    H = BLK // NSUB
    loss = jnp.float32(0.0)
    halves = []
    for h in range(NSUB):
        rh = x[h * H:(h + 1) * H, :]
        halves.append({"r": rh, "q": jnp.zeros_like(rh), "codes": []})
    for s in range(N_CODEBOOKS):
        t = table_refs[s][...]
        en = en_ref[s:s + 1, :]
        cnt = None
        for hv in halves:
            r = hv["r"]
            rn = jnp.sum(r * r, axis=1, keepdims=True)
            m2 = jax.lax.dot_general(r + r, t, (((1,), (1,)), ((), ())),
                                     preferred_element_type=jnp.float32)
            dist = (rn - m2) + en
            minv = jnp.min(dist, axis=1, keepdims=True)
            iota = jax.lax.broadcasted_iota(jnp.int32, dist.shape, 1)
            ind = jnp.min(jnp.where(dist == minv, iota, K),
                          axis=1, keepdims=True)
            hit = iota == ind
            oh16 = hit.astype(jnp.bfloat16)
            xq3 = jax.lax.dot_general(oh16, cat_refs[s][...],
                                      (((1,), (0,)), ((), ())),
                                      preferred_element_type=jnp.float32)
            xq = ((xq3[:, 0 * D:1 * D] + xq3[:, 1 * D:2 * D])
                  + xq3[:, 2 * D:3 * D])
            c = jnp.sum(hit.astype(jnp.float32), axis=0, keepdims=True)
            cnt = c if cnt is None else cnt + c
            loss = loss + jnp.sum(minv)
            hv["q"] = hv["q"] + xq
            hv["r"] = r - xq
            hv["codes"].append(ind)
        counts_ref[s:s + 1, :] += cnt

    q_ref[...] = jnp.concatenate([hv["q"] for hv in halves], axis=0)
    codes_ref[...] = jnp.concatenate(
        [jnp.concatenate(hv["codes"], axis=1) for hv in halves], axis=0)
    loss_ref[...] = loss_ref[...] + loss

    @pl.when(step == GRID - 1)
    def _finalize():
        loss_ref[...] = (loss_ref[...] * (1.0 + BETA)
                         / (N_CODEBOOKS * BATCH * D))
        unused_ref[...] = jnp.sum(
            (counts_ref[...] == 0.0).astype(jnp.int32),
            axis=(0, 1), keepdims=True)


def kernel(x, embed_0, embed_1, embed_2, embed_3):
    table_spec = pl.BlockSpec((K, D), lambda i: (0, 0))
    q, codes, loss, unused, _counts = pl.pallas_call(
        _rvq_body,
        grid=(GRID,),
        in_specs=[
            pl.BlockSpec((BLK, D), lambda i: (i, 0)),
            table_spec, table_spec, table_spec, table_spec,
        ],
        out_specs=[
            pl.BlockSpec((BLK, D), lambda i: (i, 0)),
            pl.BlockSpec((BLK, N_CODEBOOKS), lambda i: (i, 0)),
            pl.BlockSpec((1, 1), lambda i: (0, 0)),
            pl.BlockSpec((1, 1), lambda i: (0, 0)),
            pl.BlockSpec((N_CODEBOOKS, K), lambda i: (0, 0)),
        ],
        out_shape=[
            jax.ShapeDtypeStruct((BATCH, D), jnp.float32),
            jax.ShapeDtypeStruct((BATCH, N_CODEBOOKS), jnp.int32),
            jax.ShapeDtypeStruct((1, 1), jnp.float32),
            jax.ShapeDtypeStruct((1, 1), jnp.int32),
            jax.ShapeDtypeStruct((N_CODEBOOKS, K), jnp.float32),
        ],
        scratch_shapes=([pltpu.VMEM((N_CODEBOOKS, K), jnp.float32)]
                        + [pltpu.VMEM((K, 3 * D), jnp.bfloat16)] * 4),
        compiler_params=pltpu.CompilerParams(
            dimension_semantics=("arbitrary",)),
    )(x, embed_0, embed_1, embed_2, embed_3)
    return q, loss.reshape(()), unused.reshape(()), codes


# speedup vs baseline: 1.5925x; 1.0236x over previous
import jax
import jax.numpy as jnp
from jax.experimental import pallas as pl
from jax.experimental.pallas import tpu as pltpu

N_CODEBOOKS = 4
K = 1024
D = 256
BATCH = 16384
BETA = 0.25
BLK = 1024
GRID = BATCH // BLK
NSUB = 2


def _rvq_body(x_ref, e0_ref, e1_ref, e2_ref, e3_ref,
              q_ref, codes_ref, loss_ref, unused_ref, counts_ref,
              en_ref, c0_ref, c1_ref, c2_ref, c3_ref):
    step = pl.program_id(0)
    table_refs = (e0_ref, e1_ref, e2_ref, e3_ref)
    cat_refs = (c0_ref, c1_ref, c2_ref, c3_ref)

    @pl.when(step == 0)
    def _init():
        counts_ref[...] = jnp.zeros_like(counts_ref)
        loss_ref[...] = jnp.zeros_like(loss_ref)
        unused_ref[...] = jnp.zeros_like(unused_ref)
        for s in range(N_CODEBOOKS):
            t = table_refs[s][...]
            en_ref[s:s + 1, :] = jnp.sum(t * t, axis=1, keepdims=True).T
            t1 = t.astype(jnp.bfloat16)
            r1 = t - t1.astype(jnp.float32)
            t2 = r1.astype(jnp.bfloat16)
            r2 = r1 - t2.astype(jnp.float32)
            t3 = r2.astype(jnp.bfloat16)
            cat_refs[s][:, 0 * D:1 * D] = t1
            cat_refs[s][:, 1 * D:2 * D] = t2
            cat_refs[s][:, 2 * D:3 * D] = t3

    x = x_ref[...]
    H = BLK // NSUB
    loss = jnp.float32(0.0)
    halves = []
    for h in range(NSUB):
        rh = x[h * H:(h + 1) * H, :]
        halves.append({"r": rh, "x": rh, "codes": []})
    for s in range(N_CODEBOOKS):
        t = table_refs[s][...]
        en = en_ref[s:s + 1, :]
        cnt = None
        for hv in halves:
            r = hv["r"]
            rn = jnp.sum(r * r, axis=1, keepdims=True)
            m2 = jax.lax.dot_general(r + r, t, (((1,), (1,)), ((), ())),
                                     preferred_element_type=jnp.float32)
            dist = (rn - m2) + en
            minv = jnp.min(dist, axis=1, keepdims=True)
            iota = jax.lax.broadcasted_iota(jnp.int32, dist.shape, 1)
            ind = jnp.min(jnp.where(dist == minv, iota, K),
                          axis=1, keepdims=True)
            hit = iota == ind
            oh16 = hit.astype(jnp.bfloat16)
            xq3 = jax.lax.dot_general(oh16, cat_refs[s][...],
                                      (((1,), (0,)), ((), ())),
                                      preferred_element_type=jnp.float32)
            xq = ((xq3[:, 0 * D:1 * D] + xq3[:, 1 * D:2 * D])
                  + xq3[:, 2 * D:3 * D])
            c = jnp.sum(hit.astype(jnp.float32), axis=0, keepdims=True)
            cnt = c if cnt is None else cnt + c
            loss = loss + jnp.sum(minv)
            hv["r"] = r - xq
            hv["codes"].append(ind)
        counts_ref[s:s + 1, :] += cnt

    q_ref[...] = jnp.concatenate([hv["x"] - hv["r"] for hv in halves],
                                 axis=0)
    codes_ref[...] = jnp.concatenate(
        [jnp.concatenate(hv["codes"], axis=1) for hv in halves], axis=0)
    loss_ref[...] = loss_ref[...] + loss

    @pl.when(step == GRID - 1)
    def _finalize():
        loss_ref[...] = (loss_ref[...] * (1.0 + BETA)
                         / (N_CODEBOOKS * BATCH * D))
        unused_ref[...] = jnp.sum(
            (counts_ref[...] == 0.0).astype(jnp.int32),
            axis=(0, 1), keepdims=True)


def kernel(x, embed_0, embed_1, embed_2, embed_3):
    table_spec = pl.BlockSpec((K, D), lambda i: (0, 0))
    q, codes, loss, unused, _counts = pl.pallas_call(
        _rvq_body,
        grid=(GRID,),
        in_specs=[
            pl.BlockSpec((BLK, D), lambda i: (i, 0)),
            table_spec, table_spec, table_spec, table_spec,
        ],
        out_specs=[
            pl.BlockSpec((BLK, D), lambda i: (i, 0)),
            pl.BlockSpec((BLK, N_CODEBOOKS), lambda i: (i, 0)),
            pl.BlockSpec((1, 1), lambda i: (0, 0)),
            pl.BlockSpec((1, 1), lambda i: (0, 0)),
            pl.BlockSpec((N_CODEBOOKS, K), lambda i: (0, 0)),
        ],
        out_shape=[
            jax.ShapeDtypeStruct((BATCH, D), jnp.float32),
            jax.ShapeDtypeStruct((BATCH, N_CODEBOOKS), jnp.int32),
            jax.ShapeDtypeStruct((1, 1), jnp.float32),
            jax.ShapeDtypeStruct((1, 1), jnp.int32),
            jax.ShapeDtypeStruct((N_CODEBOOKS, K), jnp.float32),
        ],
        scratch_shapes=([pltpu.VMEM((N_CODEBOOKS, K), jnp.float32)]
                        + [pltpu.VMEM((K, 3 * D), jnp.bfloat16)] * 4),
        compiler_params=pltpu.CompilerParams(
            dimension_semantics=("arbitrary",)),
    )(x, embed_0, embed_1, embed_2, embed_3)
    return q, loss.reshape(()), unused.reshape(()), codes
